# baseline (speedup 1.0000x reference)
import functools

import jax
import jax.numpy as jnp
from jax.experimental import pallas as pl
from jax.experimental.pallas import tpu as pltpu

NUM_HEADS = 12
DIM = 768
HEAD_DIM = DIM // NUM_HEADS
BQ = 512
G = 12
GW = G * HEAD_DIM


def _attend(q, k, v_ext):
    s = jax.lax.dot_general(q, k, (((1,), (1,)), ((), ())),
                            preferred_element_type=jnp.float32)
    p = jnp.exp(s).astype(jnp.bfloat16)
    o_ext = jnp.dot(p, v_ext, preferred_element_type=jnp.float32)
    D = HEAD_DIM
    return o_ext[:, :D] * (1.0 / o_ext[:, D:D + 1])


def _body(x_full_ref, x_blk_ref, wq_ref, wk_ref, wv_ref,
          bq_ref, bk_ref, bv_ref, wp_ref, bp_ref,
          out_ref, k_scr, v_scr, *, scale):
    i = pl.program_id(0)
    j = pl.program_id(1)
    D = HEAD_DIM

    bf = jnp.bfloat16

    @pl.when(i == 0)
    def _():
        xf = x_full_ref[...].astype(bf)
        k_scr[j] = (jnp.dot(xf, wk_ref[...].astype(bf),
                            preferred_element_type=jnp.float32)
                    + bk_ref[0]).astype(bf)
        vv_all = (jnp.dot(xf, wv_ref[...].astype(bf),
                          preferred_element_type=jnp.float32)
                  + bv_ref[0]).astype(bf)
        ones = jnp.ones((vv_all.shape[0], D), bf)
        for g in range(G):
            v_scr[j * G + g] = jnp.concatenate(
                [vv_all[:, g * D:(g + 1) * D], ones], axis=1)

    qq = ((jnp.dot(x_blk_ref[...].astype(bf), wq_ref[...].astype(bf),
                   preferred_element_type=jnp.float32)
           + bq_ref[0]) * scale).astype(bf)
    kk = k_scr[j]
    os = [_attend(qq[:, g * D:(g + 1) * D], kk[:, g * D:(g + 1) * D],
                  v_scr[j * G + g]) for g in range(G)]
    o = jnp.concatenate(os, axis=1).astype(bf)
    contrib = jnp.dot(o, wp_ref[...].astype(bf),
                      preferred_element_type=jnp.float32)

    @pl.when(j == 0)
    def _():
        out_ref[...] = contrib + bp_ref[...]

    @pl.when(j > 0)
    def _():
        out_ref[...] += contrib


@jax.jit
def kernel(x, W_qkv, b_qkv, W_proj, b_proj):
    B, N, C = x.shape
    H, D = NUM_HEADS, HEAD_DIM
    NG = H // G
    scale = D ** -0.5
    x2 = x.reshape(N, C)
    b_qkv3 = b_qkv.reshape(3 * NG, 1, GW)
    bp = b_proj.reshape(1, C)

    nq = N // BQ
    out = pl.pallas_call(
        functools.partial(_body, scale=scale),
        grid=(nq, NG),
        in_specs=[
            pl.BlockSpec((N, C), lambda i, j: (0, 0)),
            pl.BlockSpec((BQ, C), lambda i, j: (i, 0)),
            pl.BlockSpec((C, GW), lambda i, j: (0, j)),
            pl.BlockSpec((C, GW), lambda i, j: (0, NG + j)),
            pl.BlockSpec((C, GW), lambda i, j: (0, 2 * NG + j)),
            pl.BlockSpec((1, 1, GW), lambda i, j: (j, 0, 0)),
            pl.BlockSpec((1, 1, GW), lambda i, j: (NG + j, 0, 0)),
            pl.BlockSpec((1, 1, GW), lambda i, j: (2 * NG + j, 0, 0)),
            pl.BlockSpec((GW, C), lambda i, j: (j, 0)),
            pl.BlockSpec((1, C), lambda i, j: (0, 0)),
        ],
        out_specs=pl.BlockSpec((BQ, C), lambda i, j: (i, 0)),
        out_shape=jax.ShapeDtypeStruct((N, C), jnp.float32),
        scratch_shapes=[
            pltpu.VMEM((NG, N, GW), jnp.bfloat16),
            pltpu.VMEM((H, N, 2 * D), jnp.bfloat16),
        ],
        compiler_params=pltpu.CompilerParams(
            dimension_semantics=("arbitrary", "arbitrary"),
        ),
    )(x2, x2, W_qkv, W_qkv, W_qkv, b_qkv3, b_qkv3, b_qkv3, W_proj, bp)
    return out.reshape(B, N, C)
